# Initial kernel scaffold; baseline (speedup 1.0000x reference)
import jax
import jax.numpy as jnp
from jax.experimental import pallas as pl


def kernel(x):
    raise NotImplementedError("write your pallas kernel here")


# speedup vs baseline: 29.0010x; 29.0010x over previous
import jax
import jax.numpy as jnp
from jax.experimental import pallas as pl
from jax.experimental.pallas import tpu as pltpu

_PCT = 95
_MIN_SCALE = 1e-06
_MAX_SCALE = 1000000.0
_INF_BITS = 0x7F800000


def _body(x_ref, o_ref, p_ref):
    x = x_ref[:]
    a = jnp.abs(x)
    mask = a > 1e-08
    n0 = jnp.sum(mask.astype(jnp.int32))
    n0f = n0.astype(jnp.float32)
    s = jnp.sum(jnp.where(mask, a, 0.0))
    mean = s / jnp.maximum(n0f, 1.0)
    d = a - mean
    ss = jnp.sum(jnp.where(mask, d * d, 0.0))
    var = ss / jnp.maximum(n0f - 1.0, 1.0)
    std = jnp.sqrt(var)
    refined = mask & (jnp.abs(d) <= 3.0 * std)
    nr = jnp.sum(refined.astype(jnp.int32))
    use_refined = (n0 > 10) & (nr > 0)
    n = jnp.where(use_refined, nr, n0)
    k = jnp.clip((_PCT * n) // 100, 0, n - 1)
    r = k + 1
    sel = (refined & use_refined) | (mask & jnp.logical_not(use_refined))
    bits = jax.lax.bitcast_convert_type(a, jnp.int32)
    p_ref[:] = jnp.where(sel, bits, _INF_BITS)

    def round_fn(i, ans):
        t = ans | (1 << (30 - i))
        c = jnp.sum((p_ref[:] < t).astype(jnp.int32))
        return jnp.where(c >= r, ans, t)

    ans = jax.lax.fori_loop(0, 31, round_fn, jnp.int32(0))
    val = jax.lax.bitcast_convert_type(ans, jnp.float32)
    val = jnp.where(n == 0, 1.0, val)
    value = jnp.clip(val, _MIN_SCALE, _MAX_SCALE)
    value = jnp.where(n0 == 0, 1.0, value)
    value = jnp.clip(value, _MIN_SCALE, _MAX_SCALE)
    o_ref[:] = x / (value + 1e-08)


def kernel(x):
    return pl.pallas_call(
        _body,
        out_shape=jax.ShapeDtypeStruct(x.shape, x.dtype),
        scratch_shapes=[pltpu.VMEM(x.shape, jnp.int32)],
    )(x)


# speedup vs baseline: 40.2273x; 1.3871x over previous
import jax
import jax.numpy as jnp
from jax.experimental import pallas as pl
from jax.experimental.pallas import tpu as pltpu

_PCT = 95
_MIN_SCALE = 1e-06
_MAX_SCALE = 1000000.0
_INF_BITS = 0x7F800000


def _body(x_ref, o_ref, p_ref):
    x = x_ref[:]
    a = jnp.abs(x)
    mask = a > 1e-08
    n0 = jnp.sum(mask.astype(jnp.int32))
    n0f = n0.astype(jnp.float32)
    s = jnp.sum(jnp.where(mask, a, 0.0))
    mean = s / jnp.maximum(n0f, 1.0)
    d = a - mean
    ss = jnp.sum(jnp.where(mask, d * d, 0.0))
    var = ss / jnp.maximum(n0f - 1.0, 1.0)
    std = jnp.sqrt(var)
    refined = mask & (jnp.abs(d) <= 3.0 * std)
    nr = jnp.sum(refined.astype(jnp.int32))
    use_refined = (n0 > 10) & (nr > 0)
    n = jnp.where(use_refined, nr, n0)
    k = jnp.clip((_PCT * n) // 100, 0, n - 1)
    r = k + 1
    sel = (refined & use_refined) | (mask & jnp.logical_not(use_refined))
    bits = jax.lax.bitcast_convert_type(a, jnp.int32)
    p_ref[:] = jnp.where(sel, bits, _INF_BITS)

    c30 = jnp.sum((p_ref[:] < (1 << 30)).astype(jnp.int32))
    ans0 = jnp.where(c30 >= r, 0, 1 << 30)

    def round_fn(i, ans):
        s = 2 * (14 - i)
        p = p_ref[:]
        t1 = ans | (1 << s)
        t2 = ans | (2 << s)
        t3 = ans | (3 << s)
        c1 = jnp.sum((p < t1).astype(jnp.int32))
        c2 = jnp.sum((p < t2).astype(jnp.int32))
        c3 = jnp.sum((p < t3).astype(jnp.int32))
        b = (
            (c1 < r).astype(jnp.int32)
            + (c2 < r).astype(jnp.int32)
            + (c3 < r).astype(jnp.int32)
        )
        return ans | (b << s)

    ans = jax.lax.fori_loop(0, 15, round_fn, ans0.astype(jnp.int32))
    val = jax.lax.bitcast_convert_type(ans, jnp.float32)
    val = jnp.where(n == 0, 1.0, val)
    value = jnp.clip(val, _MIN_SCALE, _MAX_SCALE)
    value = jnp.where(n0 == 0, 1.0, value)
    value = jnp.clip(value, _MIN_SCALE, _MAX_SCALE)
    o_ref[:] = x / (value + 1e-08)


def kernel(x):
    return pl.pallas_call(
        _body,
        out_shape=jax.ShapeDtypeStruct(x.shape, x.dtype),
        scratch_shapes=[pltpu.VMEM(x.shape, jnp.int32)],
    )(x)
